# baseline (speedup 1.0000x reference)
import jax
import jax.numpy as jnp
from jax.experimental import pallas as pl
from jax.experimental.pallas import tpu as pltpu

_B, _H, _W = 8, 720, 1280
_ROWS = 720
_NCHUNK = _H // _ROWS


def _wauc_kernel(inp_ref, tgt_ref, out_ref, nacc_ref, macc_ref):
    b = pl.program_id(0)
    j = pl.program_id(1)

    @pl.when((b == 0) & (j == 0))
    def _():
        nacc_ref[...] = jnp.zeros_like(nacc_ref)
        macc_ref[...] = jnp.zeros_like(macc_ref)

    ix = inp_ref[0, 0]
    iy = inp_ref[0, 1]
    tx = tgt_ref[0, 0]
    ty = tgt_ref[0, 1]
    m = tgt_ref[0, 2]

    dx = tx - ix
    dy = ty - iy
    dist = jnp.sqrt(dx * dx + dy * dy)
    k = jnp.clip(jnp.ceil(dist * 20.0), 0.0, 100.0)
    suf = ((101.0 - k) * (102.0 - k) - 2.0) * 0.005

    acc_n = suf[0:8]
    acc_m = m[0:8]
    for r in range(1, _ROWS // 8):
        acc_n = acc_n + suf[8 * r:8 * (r + 1)]
        acc_m = acc_m + m[8 * r:8 * (r + 1)]

    nacc_ref[...] += acc_n
    macc_ref[...] += acc_m

    @pl.when((b == _B - 1) & (j == _NCHUNK - 1))
    def _():
        n_err = jnp.sum(nacc_ref[...])
        mask_sum = jnp.sum(macc_ref[...])
        out_ref[0, 0] = 100.0 * n_err / (mask_sum * 51.5)


def kernel(input_flow, target_flow):
    out = pl.pallas_call(
        _wauc_kernel,
        grid=(_B, _NCHUNK),
        in_specs=[
            pl.BlockSpec((1, 2, _ROWS, _W), lambda b, j: (b, 0, j, 0)),
            pl.BlockSpec((1, 3, _ROWS, _W), lambda b, j: (b, 0, j, 0)),
        ],
        out_specs=pl.BlockSpec(memory_space=pltpu.SMEM),
        out_shape=jax.ShapeDtypeStruct((1, 1), jnp.float32),
        scratch_shapes=[
            pltpu.VMEM((8, _W), jnp.float32),
            pltpu.VMEM((8, _W), jnp.float32),
        ],
        compiler_params=pltpu.CompilerParams(
            dimension_semantics=("arbitrary", "arbitrary"),
            vmem_limit_bytes=60 * 1024 * 1024,
        ),
        name="wauc_fused",
    )(input_flow, target_flow)
    return out[0, 0]


# speedup vs baseline: 1.0101x; 1.0101x over previous
import jax
import jax.numpy as jnp
from jax.experimental import pallas as pl
from jax.experimental.pallas import tpu as pltpu

_B, _H, _W = 8, 720, 1280
_ROWS = 360
_NCHUNK = _H // _ROWS


def _wauc_kernel(inp_ref, tgt_ref, out_ref, nacc_ref, macc_ref):
    b = pl.program_id(0)
    j = pl.program_id(1)

    @pl.when((b == 0) & (j == 0))
    def _():
        nacc_ref[...] = jnp.zeros_like(nacc_ref)
        macc_ref[...] = jnp.zeros_like(macc_ref)

    ix = inp_ref[0, 0]
    iy = inp_ref[0, 1]
    tx = tgt_ref[0, 0]
    ty = tgt_ref[0, 1]
    m = tgt_ref[0, 2]

    dx = tx - ix
    dy = ty - iy
    dist = jnp.sqrt(dx * dx + dy * dy)
    k = jnp.clip(jnp.ceil(dist * 20.0), 0.0, 100.0)
    suf = ((101.0 - k) * (102.0 - k) - 2.0) * 0.005

    acc_n = suf[0:8]
    acc_m = m[0:8]
    for r in range(1, _ROWS // 8):
        acc_n = acc_n + suf[8 * r:8 * (r + 1)]
        acc_m = acc_m + m[8 * r:8 * (r + 1)]

    nacc_ref[...] += acc_n
    macc_ref[...] += acc_m

    @pl.when((b == _B - 1) & (j == _NCHUNK - 1))
    def _():
        n_err = jnp.sum(nacc_ref[...])
        mask_sum = jnp.sum(macc_ref[...])
        out_ref[0, 0] = 100.0 * n_err / (mask_sum * 51.5)


def kernel(input_flow, target_flow):
    out = pl.pallas_call(
        _wauc_kernel,
        grid=(_B, _NCHUNK),
        in_specs=[
            pl.BlockSpec((1, 2, _ROWS, _W), lambda b, j: (b, 0, j, 0)),
            pl.BlockSpec((1, 3, _ROWS, _W), lambda b, j: (b, 0, j, 0)),
        ],
        out_specs=pl.BlockSpec(memory_space=pltpu.SMEM),
        out_shape=jax.ShapeDtypeStruct((1, 1), jnp.float32),
        scratch_shapes=[
            pltpu.VMEM((8, _W), jnp.float32),
            pltpu.VMEM((8, _W), jnp.float32),
        ],
        compiler_params=pltpu.CompilerParams(
            dimension_semantics=("arbitrary", "arbitrary"),
            vmem_limit_bytes=60 * 1024 * 1024,
        ),
        name="wauc_fused",
    )(input_flow, target_flow)
    return out[0, 0]
